# baseline (speedup 1.0000x reference)
import functools

import jax
import jax.numpy as jnp
from jax import lax
from jax.experimental import pallas as pl
from jax.experimental.pallas import tpu as pltpu
from jax.experimental.pallas import tpu_sc as plsc

_B, _T, _D = 4, 2048, 1024
_NC, _NS = 2, 16
_NW = _NC * _NS
_TPW = _T // _NW
_R = 16
_NSUB = _TPW // _R
_UNROLL = 8
_CPR = _D // 16


def _sc_body(feat_hbm, sin_hbm, out_hbm, sin_v, feat_v):
    wid = lax.axis_index("s") * _NC + lax.axis_index("c")
    t0 = wid * _TPW

    def per_s(s, carry):
        row = t0 + s * _R
        pltpu.sync_copy(sin_hbm.at[pl.ds(row, _R)], sin_v)

        def per_b(b, c2):
            pltpu.sync_copy(feat_hbm.at[b, pl.ds(row, _R)], feat_v)
            for r in range(_R):
                def add_block(kk, c3, r=r):
                    for j in range(_UNROLL):
                        sl = pl.ds((kk * _UNROLL + j) * 16, 16)
                        plsc.addupdate(feat_v.at[r, sl], sin_v[r, sl])
                    return c3
                lax.fori_loop(0, _CPR // _UNROLL, add_block, 0)
            pltpu.sync_copy(feat_v, out_hbm.at[b, pl.ds(row, _R)])
            return c2

        return lax.fori_loop(0, _B, per_b, carry)

    lax.fori_loop(0, _NSUB, per_s, 0)


_sc_kernel = functools.partial(
    pl.kernel,
    out_type=jax.ShapeDtypeStruct((_B, _T, _D), jnp.float32),
    mesh=plsc.VectorSubcoreMesh(core_axis_name="c", subcore_axis_name="s"),
    scratch_types=[
        pltpu.VMEM((_R, _D), jnp.float32),
        pltpu.VMEM((_R, _D), jnp.float32),
    ],
)(_sc_body)


def kernel(features, sinusoids):
    return _sc_kernel(features, sinusoids)


# speedup vs baseline: 1.9546x; 1.9546x over previous
import functools

import jax
import jax.numpy as jnp
from jax import lax
from jax.experimental import pallas as pl
from jax.experimental.pallas import tpu as pltpu
from jax.experimental.pallas import tpu_sc as plsc

_B, _T, _D = 4, 2048, 1024
_NC, _NS = 2, 16
_NW = _NC * _NS
_TPW = _T // _NW
_R = 16
_NSUB = _TPW // _R
_UNROLL = 8
_CPR = _D // 16


def _sc_body(feat_hbm, sin_hbm, out_hbm, sin_v, feat_v):
    wid = lax.axis_index("s") * _NC + lax.axis_index("c")
    t0 = wid * _TPW

    def per_s(s, carry):
        row = t0 + s * _R
        pltpu.sync_copy(sin_hbm.at[pl.ds(row, _R)], sin_v)

        def per_b(b, c2):
            pltpu.sync_copy(feat_hbm.at[b, pl.ds(row, _R)], feat_v)
            for r in range(_R):
                @plsc.parallel_loop(0, _CPR, 1, unroll=_UNROLL)
                def add_col(c, r=r):
                    sl = pl.ds(c * 16, 16)
                    plsc.addupdate(feat_v.at[r, sl], sin_v[r, sl])
            pltpu.sync_copy(feat_v, out_hbm.at[b, pl.ds(row, _R)])
            return c2

        return lax.fori_loop(0, _B, per_b, carry)

    lax.fori_loop(0, _NSUB, per_s, 0)


_sc_kernel = functools.partial(
    pl.kernel,
    out_type=jax.ShapeDtypeStruct((_B, _T, _D), jnp.float32),
    mesh=plsc.VectorSubcoreMesh(core_axis_name="c", subcore_axis_name="s"),
    scratch_types=[
        pltpu.VMEM((_R, _D), jnp.float32),
        pltpu.VMEM((_R, _D), jnp.float32),
    ],
)(_sc_body)


def kernel(features, sinusoids):
    return _sc_kernel(features, sinusoids)


# speedup vs baseline: 2.7120x; 1.3874x over previous
import functools

import jax
import jax.numpy as jnp
from jax import lax
from jax.experimental import pallas as pl
from jax.experimental.pallas import tpu as pltpu
from jax.experimental.pallas import tpu_sc as plsc

_B, _T, _D = 4, 2048, 1024
_NC, _NS = 2, 16
_NW = _NC * _NS
_TPW = _T // _NW
_R = 16
_NSUB = _TPW // _R
_NITEMS = _NSUB * _B
_UNROLL = 8
_CPR = _D // 16


def _sc_body(feat_hbm, sin_hbm, out_hbm,
             sin_v, fb0, fb1, si0, si1, so0, so1):
    wid = lax.axis_index("s") * _NC + lax.axis_index("c")
    t0 = wid * _TPW

    def item_coords(i):
        s = lax.shift_right_logical(i, (_B - 1).bit_length())
        b = lax.bitwise_and(i, _B - 1)
        return b, t0 + s * _R

    b0, row0 = item_coords(0)
    pltpu.async_copy(feat_hbm.at[b0, pl.ds(row0, _R)], fb0, si0)

    def stage(i, b, row, cur, si_cur, so_cur, nxt, si_nxt, so_nxt):
        @pl.when(i > 0)
        def _():
            pltpu.make_async_copy(nxt, out_hbm.at[0, pl.ds(0, _R)], so_nxt).wait()

        @pl.when(i < _NITEMS - 1)
        def _():
            b2, row2 = item_coords(i + 1)
            pltpu.async_copy(feat_hbm.at[b2, pl.ds(row2, _R)], nxt, si_nxt)

        pltpu.make_async_copy(feat_hbm.at[0, pl.ds(0, _R)], cur, si_cur).wait()

        for r in range(_R):
            @plsc.parallel_loop(0, _CPR, 1, unroll=_UNROLL)
            def add_col(c, r=r):
                sl = pl.ds(c * 16, 16)
                plsc.addupdate(cur.at[r, sl], sin_v[r, sl])

        pltpu.async_copy(cur, out_hbm.at[b, pl.ds(row, _R)], so_cur)

    def per_item(i, carry):
        b, row = item_coords(i)

        @pl.when(b == 0)
        def _():
            pltpu.sync_copy(sin_hbm.at[pl.ds(row, _R)], sin_v)

        p = lax.bitwise_and(i, 1)

        @pl.when(p == 0)
        def _():
            stage(i, b, row, fb0, si0, so0, fb1, si1, so1)

        @pl.when(p == 1)
        def _():
            stage(i, b, row, fb1, si1, so1, fb0, si0, so0)

        return carry

    lax.fori_loop(0, _NITEMS, per_item, 0)

    if (_NITEMS - 1) % 2 == 0:
        pltpu.make_async_copy(fb0, out_hbm.at[0, pl.ds(0, _R)], so0).wait()
    else:
        pltpu.make_async_copy(fb1, out_hbm.at[0, pl.ds(0, _R)], so1).wait()


_sc_kernel = functools.partial(
    pl.kernel,
    out_type=jax.ShapeDtypeStruct((_B, _T, _D), jnp.float32),
    mesh=plsc.VectorSubcoreMesh(core_axis_name="c", subcore_axis_name="s"),
    scratch_types=[
        pltpu.VMEM((_R, _D), jnp.float32),
        pltpu.VMEM((_R, _D), jnp.float32),
        pltpu.VMEM((_R, _D), jnp.float32),
        pltpu.SemaphoreType.DMA,
        pltpu.SemaphoreType.DMA,
        pltpu.SemaphoreType.DMA,
        pltpu.SemaphoreType.DMA,
    ],
)(_sc_body)


def kernel(features, sinusoids):
    return _sc_kernel(features, sinusoids)
